# baseline (speedup 1.0000x reference)
import functools

import jax
import jax.numpy as jnp
from jax import lax
from jax.experimental import pallas as pl
from jax.experimental.pallas import tpu as pltpu
from jax.experimental.pallas import tpu_sc as plsc

N = 10000
E = 320000
FIN = 128
HID = 128
CLS = 64

NC = 2
NS = 16
NW = NC * NS
CH = 128
ROWS = 2560
EPAD = ROWS * CH - E
NP = 10240
NACC = 10112
ZR = NP // NS
ZRA = NACC // NS

BR = 512
GRID = NP // BR

_mesh = plsc.VectorSubcoreMesh(
    core_axis_name="c", subcore_axis_name="s", num_cores=NC, num_subcores=NS
)


NCH = ROWS // NW
NBUF = 8
NSTEP = NCH // NBUF


def _worker_ids():
    c = lax.axis_index("c")
    s = lax.axis_index("s")
    return c, s, c * NS + s


def _load_idx(src2, dst2, srcall, dstall, w):
    base = pl.multiple_of(w * NCH, 8)
    pltpu.sync_copy(src2.at[pl.ds(base, NCH)], srcall)
    pltpu.sync_copy(dst2.at[pl.ds(base, NCH)], dstall)


def _fill_zeros_2d(ref, nrows, ncols):
    z = jnp.zeros((16,), jnp.float32)

    def body(i, carry):
        r = i // (ncols // 16)
        col = (i % (ncols // 16)) * 16
        ref[r, pl.ds(col, 16)] = z
        return carry

    lax.fori_loop(0, nrows * ncols // 16, body, 0)


def _fill_1d(ref, n, value):
    v = jnp.full((16,), value, jnp.float32)

    def body(i, carry):
        ref[pl.ds(i * 16, 16)] = v
        return carry

    lax.fori_loop(0, n // 16, body, 0)


@functools.partial(
    pl.kernel,
    out_type=jax.ShapeDtypeStruct((NC, 2, NP), jnp.float32),
    mesh=_mesh,
    scratch_types=[
        pltpu.VMEM((NCH, CH), jnp.int32),
        pltpu.VMEM((NCH, CH), jnp.int32),
        pltpu.VMEM((CH,), jnp.float32),
        pltpu.VMEM((ZR,), jnp.float32),
        pltpu.VMEM_SHARED((NP,), jnp.float32),
        pltpu.VMEM_SHARED((NP,), jnp.float32),
    ]
    + [pltpu.SemaphoreType.DMA] * (2 * NBUF),
)
def _deg_kernel(src2, dst2, out, srcall, dstall, onesv, zbuf, acc_out,
                acc_in, *sems):
    sa, sb = sems[:NBUF], sems[NBUF:]
    c, s, w = _worker_ids()
    _load_idx(src2, dst2, srcall, dstall, w)
    _fill_1d(zbuf, ZR, 0.0)
    _fill_1d(onesv, CH, 1.0)
    zb = s * ZR
    pltpu.sync_copy(zbuf, acc_out.at[pl.ds(zb, ZR)])
    pltpu.sync_copy(zbuf, acc_in.at[pl.ds(zb, ZR)])
    plsc.subcore_barrier()

    def step(t, carry):
        for j in range(NBUF):
            k = t * NBUF + j
            pltpu.async_copy(onesv, acc_out.at[srcall.at[k]], sa[j], add=True)
            pltpu.async_copy(onesv, acc_in.at[dstall.at[k]], sb[j], add=True)

            @pl.when(t > 0)
            def _():
                pltpu.make_async_copy(onesv, acc_out.at[srcall.at[0]], sa[j]).wait()
                pltpu.make_async_copy(onesv, acc_in.at[dstall.at[0]], sb[j]).wait()

        return carry

    lax.fori_loop(0, NSTEP, step, 0)
    for j in range(NBUF):
        pltpu.make_async_copy(onesv, acc_out.at[srcall.at[0]], sa[j]).wait()
        pltpu.make_async_copy(onesv, acc_in.at[dstall.at[0]], sb[j]).wait()

    plsc.subcore_barrier()
    pltpu.sync_copy(acc_out.at[pl.ds(zb, ZR)], out.at[c, 0, pl.ds(zb, ZR)])
    pltpu.sync_copy(acc_in.at[pl.ds(zb, ZR)], out.at[c, 1, pl.ds(zb, ZR)])


def _zero_acc(acc, zrows, s, nr):
    done = 0
    while done < nr:
        n = min(CH, nr - done)
        pltpu.sync_copy(
            zrows.at[pl.ds(0, n), :], acc.at[pl.ds(s * nr + done, n), :]
        )
        done += n


def _make_agg_kernel_slim(F):
    @functools.partial(
        pl.kernel,
        out_type=jax.ShapeDtypeStruct((NC, NACC, F), jnp.float32),
        mesh=_mesh,
        compiler_params=pltpu.CompilerParams(use_tc_tiling_on_sc=True),
        scratch_types=[pltpu.VMEM_SHARED((NACC, F), jnp.float32)]
        + [pltpu.VMEM((CH,), jnp.int32)] * 6
        + [pltpu.VMEM((CH, F), jnp.float32)] * 3
        + [pltpu.SemaphoreType.DMA] * 12,
    )
    def _agg_kernel(table, src2, dst2, out, acc, *ring):
        srcv, dstv = ring[0:3], ring[3:6]
        rows = ring[6:9]
        gs, ss = ring[9:12], ring[12:15]
        is_, id_ = ring[15:18], ring[18:21]
        c, s, w = _worker_ids()
        _fill_zeros_2d(rows[0], CH, F)
        _zero_acc(acc, rows[0], s, ZRA)
        plsc.subcore_barrier()

        def is_issue(k, b):
            pltpu.async_copy(src2.at[w * NCH + k], srcv[b], is_[b])

        def is_wait(b):
            pltpu.make_async_copy(src2.at[0], srcv[b], is_[b]).wait()

        def id_issue(k, b):
            pltpu.async_copy(dst2.at[w * NCH + k], dstv[b], id_[b])

        def id_wait(b):
            pltpu.make_async_copy(dst2.at[0], dstv[b], id_[b]).wait()

        def g_issue(b):
            pltpu.async_copy(table.at[srcv[b]], rows[b], gs[b])

        def g_wait(b):
            pltpu.make_async_copy(table.at[srcv[b]], rows[b], gs[b]).wait()

        def s_issue(b):
            pltpu.async_copy(rows[b], acc.at[dstv[b]], ss[b], add=True)

        def s_wait(b):
            pltpu.make_async_copy(rows[b], acc.at[dstv[b]], ss[b]).wait()

        def it(k, b, first=False, n2=True, n3=True):
            b2 = (b + 2) % 3
            id_wait(b)
            g_wait(b)
            s_issue(b)
            if not first:
                s_wait(b2)
            if n2:
                id_issue(k + 2, b2)
                is_wait(b2)
                g_issue(b2)
            if n3:
                is_issue(k + 3, b)

        is_issue(0, 0)
        id_issue(0, 0)
        is_issue(1, 1)
        id_issue(1, 1)
        is_issue(2, 2)
        is_wait(0)
        g_issue(0)
        is_wait(1)
        g_issue(1)
        it(0, 0, first=True)
        it(1, 1)

        def step(t, carry):
            for j in range(3):
                it(2 + 3 * t + j, (2 + j) % 3)
            return carry

        lax.fori_loop(0, (NCH - 5) // 3, step, 0)
        it(NCH - 3, (NCH - 3) % 3, n3=False)
        it(NCH - 2, (NCH - 2) % 3, n2=False, n3=False)
        it(NCH - 1, (NCH - 1) % 3, n2=False, n3=False)
        s_wait((NCH - 1) % 3)
        plsc.subcore_barrier()
        pltpu.sync_copy(
            acc.at[pl.ds(s * ZRA, ZRA), :], out.at[c, pl.ds(s * ZRA, ZRA), :]
        )

    return _agg_kernel


def _make_agg_kernel(F):
    if F == FIN:
        return _make_agg_kernel_slim(F)

    @functools.partial(
        pl.kernel,
        out_type=jax.ShapeDtypeStruct((NC, NACC, F), jnp.float32),
        mesh=_mesh,
        compiler_params=pltpu.CompilerParams(use_tc_tiling_on_sc=(F % 128 == 0)),
        scratch_types=[
            pltpu.VMEM((NCH, CH), jnp.int32),
            pltpu.VMEM((NCH, CH), jnp.int32),
            pltpu.VMEM_SHARED((NACC, F), jnp.float32),
        ]
        + [pltpu.VMEM((CH, F), jnp.float32)] * NBUF
        + [pltpu.SemaphoreType.DMA] * (2 * NBUF),
    )
    def _agg_kernel(table, src2, dst2, out, srcall, dstall, acc, *ring):
        rows = ring[:NBUF]
        gs = ring[NBUF:2 * NBUF]
        ss = ring[2 * NBUF:]
        c, s, w = _worker_ids()
        _load_idx(src2, dst2, srcall, dstall, w)
        _fill_zeros_2d(rows[0], CH, F)
        _zero_acc(acc, rows[0], s, ZRA)
        plsc.subcore_barrier()

        def g_issue(k, b):
            pltpu.async_copy(table.at[srcall.at[k]], rows[b], gs[b])

        def g_wait(b):
            pltpu.make_async_copy(table.at[srcall.at[0]], rows[b], gs[b]).wait()

        def s_issue(k, b):
            pltpu.async_copy(rows[b], acc.at[dstall.at[k]], ss[b], add=True)

        def s_wait(b):
            pltpu.make_async_copy(rows[b], acc.at[dstall.at[0]], ss[b]).wait()

        def iter_chunk(k, j, ss_wait, g_next):
            g_wait(j)
            s_issue(k, j)
            bn = (j + 3) % NBUF
            if ss_wait:
                s_wait(bn)
            if g_next:
                g_issue(k + 3, bn)

        for b in range(3):
            g_issue(b, b)
        for j in range(NBUF):
            iter_chunk(j, j, ss_wait=(j >= NBUF - 3), g_next=True)

        def step(t, carry):
            for j in range(NBUF):
                iter_chunk(t * NBUF + j, j, ss_wait=True, g_next=True)
            return carry

        lax.fori_loop(1, NSTEP - 1, step, 0)
        for j in range(NBUF):
            iter_chunk((NSTEP - 1) * NBUF + j, j, ss_wait=(j < NBUF - 3),
                       g_next=(j < NBUF - 3))
        for b in range(NBUF):
            s_wait(b)

        plsc.subcore_barrier()
        pltpu.sync_copy(
            acc.at[pl.ds(s * ZRA, ZRA), :], out.at[c, pl.ds(s * ZRA, ZRA), :]
        )

    return _agg_kernel


_agg128 = _make_agg_kernel(FIN)
_agg64 = _make_agg_kernel(CLS)


def _prep_body(f_ref, dp_ref, x1_ref, ns_ref, nd_ref):
    dout = dp_ref[0, 0] + dp_ref[1, 0]
    din = dp_ref[0, 1] + dp_ref[1, 1]
    ns = lax.rsqrt(jnp.maximum(dout, 1.0)).reshape(BR, 1)
    nd = lax.rsqrt(jnp.maximum(din, 1.0)).reshape(BR, 1)
    x1_ref[...] = f_ref[...] * ns
    ns_ref[...] = ns
    nd_ref[...] = nd


def _mid_body(p_ref, nd_ref, ns_ref, w1_ref, b1_ref, w2_ref, t2_ref):
    agg = (p_ref[0] + p_ref[1]) * nd_ref[...]
    h = jnp.dot(agg, w1_ref[...], preferred_element_type=jnp.float32)
    h = jnp.maximum(h + b1_ref[...], 0.0)
    t2_ref[...] = jnp.dot(
        h * ns_ref[...], w2_ref[...], preferred_element_type=jnp.float32
    )


def _fin_body(p_ref, nd_ref, b2_ref, out_ref):
    sc = (p_ref[0] + p_ref[1]) * nd_ref[...] + b2_ref[...]
    m = jnp.max(sc, axis=1, keepdims=True)
    lse = jnp.log(jnp.sum(jnp.exp(sc - m), axis=1, keepdims=True)) + m
    out_ref[...] = sc - lse


def _row_block(shape_minor):
    return pl.BlockSpec((BR,) + shape_minor, lambda i: (i,) + (0,) * len(shape_minor))


def kernel(features, edge_index, W1, b1, W2, b2):
    pad = (N + jnp.arange(EPAD, dtype=jnp.int32) % (NACC - N))
    src2 = jnp.concatenate([edge_index[0], pad]).reshape(ROWS, CH)
    dst2 = jnp.concatenate([edge_index[1], pad]).reshape(ROWS, CH)

    deg = _deg_kernel(src2, dst2)

    x1, ns, nd = pl.pallas_call(
        _prep_body,
        grid=(GRID,),
        in_specs=[
            _row_block((FIN,)),
            pl.BlockSpec((NC, 2, BR), lambda i: (0, 0, i)),
        ],
        out_specs=[_row_block((FIN,)), _row_block((1,)), _row_block((1,))],
        out_shape=[
            jax.ShapeDtypeStruct((NP, FIN), jnp.float32),
            jax.ShapeDtypeStruct((NP, 1), jnp.float32),
            jax.ShapeDtypeStruct((NP, 1), jnp.float32),
        ],
    )(features, deg)

    p1 = _agg128(x1, src2, dst2)

    t2 = pl.pallas_call(
        _mid_body,
        grid=(GRID,),
        in_specs=[
            pl.BlockSpec((NC, BR, FIN), lambda i: (0, i, 0)),
            _row_block((1,)),
            _row_block((1,)),
            pl.BlockSpec((FIN, HID), lambda i: (0, 0)),
            pl.BlockSpec((1, HID), lambda i: (0, 0)),
            pl.BlockSpec((HID, CLS), lambda i: (0, 0)),
        ],
        out_specs=_row_block((CLS,)),
        out_shape=jax.ShapeDtypeStruct((NP, CLS), jnp.float32),
    )(p1, nd, ns, W1, b1.reshape(1, HID), W2)

    p2 = _agg64(t2, src2, dst2)

    out = pl.pallas_call(
        _fin_body,
        grid=(GRID,),
        in_specs=[
            pl.BlockSpec((NC, BR, CLS), lambda i: (0, i, 0)),
            _row_block((1,)),
            pl.BlockSpec((1, CLS), lambda i: (0, 0)),
        ],
        out_specs=_row_block((CLS,)),
        out_shape=jax.ShapeDtypeStruct((NP, CLS), jnp.float32),
    )(p2, nd, b2.reshape(1, CLS))

    return out[:N]


# speedup vs baseline: 1.0183x; 1.0183x over previous
import functools

import jax
import jax.numpy as jnp
from jax import lax
from jax.experimental import pallas as pl
from jax.experimental.pallas import tpu as pltpu
from jax.experimental.pallas import tpu_sc as plsc

N = 10000
E = 320000
FIN = 128
HID = 128
CLS = 64

NC = 2
NS = 16
NW = NC * NS
CH = 128
ROWS = 2560
EPAD = ROWS * CH - E
NP = 10240
NACC = 10112
ZR = NP // NS
ZRA = NACC // NS

BR = 512
GRID = NP // BR

_mesh = plsc.VectorSubcoreMesh(
    core_axis_name="c", subcore_axis_name="s", num_cores=NC, num_subcores=NS
)


NCH = ROWS // NW
NBUF = 5
NSTEP = NCH // NBUF


def _worker_ids():
    c = lax.axis_index("c")
    s = lax.axis_index("s")
    return c, s, c * NS + s


def _load_idx(src2, dst2, srcall, dstall, w):
    base = pl.multiple_of(w * NCH, 8)
    pltpu.sync_copy(src2.at[pl.ds(base, NCH)], srcall)
    pltpu.sync_copy(dst2.at[pl.ds(base, NCH)], dstall)


def _fill_zeros_2d(ref, nrows, ncols):
    z = jnp.zeros((16,), jnp.float32)

    def body(i, carry):
        r = i // (ncols // 16)
        col = (i % (ncols // 16)) * 16
        ref[r, pl.ds(col, 16)] = z
        return carry

    lax.fori_loop(0, nrows * ncols // 16, body, 0)


def _fill_1d(ref, n, value):
    v = jnp.full((16,), value, jnp.float32)

    def body(i, carry):
        ref[pl.ds(i * 16, 16)] = v
        return carry

    lax.fori_loop(0, n // 16, body, 0)


@functools.partial(
    pl.kernel,
    out_type=jax.ShapeDtypeStruct((NC, 2, NP), jnp.float32),
    mesh=_mesh,
    compiler_params=pltpu.CompilerParams(skip_device_barrier=True),
    scratch_types=[
        pltpu.VMEM((NCH, CH), jnp.int32),
        pltpu.VMEM((NCH, CH), jnp.int32),
        pltpu.VMEM((CH,), jnp.float32),
        pltpu.VMEM((ZR,), jnp.float32),
        pltpu.VMEM_SHARED((NP,), jnp.float32),
        pltpu.VMEM_SHARED((NP,), jnp.float32),
    ]
    + [pltpu.SemaphoreType.DMA] * (2 * NBUF),
)
def _deg_kernel(src2, dst2, out, srcall, dstall, onesv, zbuf, acc_out,
                acc_in, *sems):
    sa, sb = sems[:NBUF], sems[NBUF:]
    c, s, w = _worker_ids()
    _load_idx(src2, dst2, srcall, dstall, w)
    _fill_1d(zbuf, ZR, 0.0)
    _fill_1d(onesv, CH, 1.0)
    zb = s * ZR
    pltpu.sync_copy(zbuf, acc_out.at[pl.ds(zb, ZR)])
    pltpu.sync_copy(zbuf, acc_in.at[pl.ds(zb, ZR)])
    plsc.subcore_barrier()

    def step(t, carry):
        for j in range(NBUF):
            k = t * NBUF + j
            pltpu.async_copy(onesv, acc_out.at[srcall.at[k]], sa[j], add=True)
            pltpu.async_copy(onesv, acc_in.at[dstall.at[k]], sb[j], add=True)

            @pl.when(t > 0)
            def _():
                pltpu.make_async_copy(onesv, acc_out.at[srcall.at[0]], sa[j]).wait()
                pltpu.make_async_copy(onesv, acc_in.at[dstall.at[0]], sb[j]).wait()

        return carry

    lax.fori_loop(0, NSTEP, step, 0)
    for j in range(NBUF):
        pltpu.make_async_copy(onesv, acc_out.at[srcall.at[0]], sa[j]).wait()
        pltpu.make_async_copy(onesv, acc_in.at[dstall.at[0]], sb[j]).wait()

    plsc.subcore_barrier()
    pltpu.sync_copy(acc_out.at[pl.ds(zb, ZR)], out.at[c, 0, pl.ds(zb, ZR)])
    pltpu.sync_copy(acc_in.at[pl.ds(zb, ZR)], out.at[c, 1, pl.ds(zb, ZR)])


def _zero_acc(acc, zrows, s, nr):
    done = 0
    while done < nr:
        n = min(CH, nr - done)
        pltpu.sync_copy(
            zrows.at[pl.ds(0, n), :], acc.at[pl.ds(s * nr + done, n), :]
        )
        done += n


def _make_agg_kernel_slim(F):
    @functools.partial(
        pl.kernel,
        out_type=jax.ShapeDtypeStruct((NC, NACC, F), jnp.float32),
        mesh=_mesh,
        compiler_params=pltpu.CompilerParams(
            use_tc_tiling_on_sc=True, skip_device_barrier=True
        ),
        scratch_types=[pltpu.VMEM_SHARED((NACC, F), jnp.float32)]
        + [pltpu.VMEM((CH,), jnp.int32)] * 6
        + [pltpu.VMEM((CH, F), jnp.float32)] * 3
        + [pltpu.SemaphoreType.DMA] * 12,
    )
    def _agg_kernel(table, src2, dst2, out, acc, *ring):
        srcv, dstv = ring[0:3], ring[3:6]
        rows = ring[6:9]
        gs, ss = ring[9:12], ring[12:15]
        is_, id_ = ring[15:18], ring[18:21]
        c, s, w = _worker_ids()
        _fill_zeros_2d(rows[0], CH, F)
        _zero_acc(acc, rows[0], s, ZRA)
        plsc.subcore_barrier()

        def is_issue(k, b):
            pltpu.async_copy(src2.at[w * NCH + k], srcv[b], is_[b])

        def is_wait(b):
            pltpu.make_async_copy(src2.at[0], srcv[b], is_[b]).wait()

        def id_issue(k, b):
            pltpu.async_copy(dst2.at[w * NCH + k], dstv[b], id_[b])

        def id_wait(b):
            pltpu.make_async_copy(dst2.at[0], dstv[b], id_[b]).wait()

        def g_issue(b):
            pltpu.async_copy(table.at[srcv[b]], rows[b], gs[b])

        def g_wait(b):
            pltpu.make_async_copy(table.at[srcv[b]], rows[b], gs[b]).wait()

        def s_issue(b):
            pltpu.async_copy(rows[b], acc.at[dstv[b]], ss[b], add=True)

        def s_wait(b):
            pltpu.make_async_copy(rows[b], acc.at[dstv[b]], ss[b]).wait()

        def it(k, b, first=False, n2=True, n3=True):
            b2 = (b + 2) % 3
            id_wait(b)
            g_wait(b)
            s_issue(b)
            if not first:
                s_wait(b2)
            if n2:
                id_issue(k + 2, b2)
                is_wait(b2)
                g_issue(b2)
            if n3:
                is_issue(k + 3, b)

        is_issue(0, 0)
        id_issue(0, 0)
        is_issue(1, 1)
        id_issue(1, 1)
        is_issue(2, 2)
        is_wait(0)
        g_issue(0)
        is_wait(1)
        g_issue(1)
        it(0, 0, first=True)
        it(1, 1)

        def step(t, carry):
            for j in range(3):
                it(2 + 3 * t + j, (2 + j) % 3)
            return carry

        lax.fori_loop(0, (NCH - 5) // 3, step, 0)
        it(NCH - 3, (NCH - 3) % 3, n3=False)
        it(NCH - 2, (NCH - 2) % 3, n2=False, n3=False)
        it(NCH - 1, (NCH - 1) % 3, n2=False, n3=False)
        s_wait((NCH - 1) % 3)
        plsc.subcore_barrier()
        pltpu.sync_copy(
            acc.at[pl.ds(s * ZRA, ZRA), :], out.at[c, pl.ds(s * ZRA, ZRA), :]
        )

    return _agg_kernel


def _make_agg_kernel(F):
    if F == FIN:
        return _make_agg_kernel_slim(F)

    @functools.partial(
        pl.kernel,
        out_type=jax.ShapeDtypeStruct((NC, NACC, F), jnp.float32),
        mesh=_mesh,
        compiler_params=pltpu.CompilerParams(
            use_tc_tiling_on_sc=(F % 128 == 0), skip_device_barrier=True
        ),
        scratch_types=[
            pltpu.VMEM((NCH, CH), jnp.int32),
            pltpu.VMEM((NCH, CH), jnp.int32),
            pltpu.VMEM_SHARED((NACC, F), jnp.float32),
        ]
        + [pltpu.VMEM((CH, F), jnp.float32)] * NBUF
        + [pltpu.SemaphoreType.DMA] * (2 * NBUF),
    )
    def _agg_kernel(table, src2, dst2, out, srcall, dstall, acc, *ring):
        rows = ring[:NBUF]
        gs = ring[NBUF:2 * NBUF]
        ss = ring[2 * NBUF:]
        c, s, w = _worker_ids()
        _load_idx(src2, dst2, srcall, dstall, w)
        _fill_zeros_2d(rows[0], CH, F)
        _zero_acc(acc, rows[0], s, ZRA)
        plsc.subcore_barrier()

        def g_issue(k, b):
            pltpu.async_copy(table.at[srcall.at[k]], rows[b], gs[b])

        def g_wait(b):
            pltpu.make_async_copy(table.at[srcall.at[0]], rows[b], gs[b]).wait()

        def s_issue(k, b):
            pltpu.async_copy(rows[b], acc.at[dstall.at[k]], ss[b], add=True)

        def s_wait(b):
            pltpu.make_async_copy(rows[b], acc.at[dstall.at[0]], ss[b]).wait()

        def iter_chunk(k, j, ss_wait, g_next):
            g_wait(j)
            s_issue(k, j)
            bn = (j + 3) % NBUF
            if ss_wait:
                s_wait(bn)
            if g_next:
                g_issue(k + 3, bn)

        for b in range(3):
            g_issue(b, b)
        for j in range(NBUF):
            iter_chunk(j, j, ss_wait=(j >= NBUF - 3), g_next=True)

        def step(t, carry):
            for j in range(NBUF):
                iter_chunk(t * NBUF + j, j, ss_wait=True, g_next=True)
            return carry

        lax.fori_loop(1, NSTEP - 1, step, 0)
        for j in range(NBUF):
            iter_chunk((NSTEP - 1) * NBUF + j, j, ss_wait=(j < NBUF - 3),
                       g_next=(j < NBUF - 3))
        for b in range(NBUF):
            s_wait(b)

        plsc.subcore_barrier()
        pltpu.sync_copy(
            acc.at[pl.ds(s * ZRA, ZRA), :], out.at[c, pl.ds(s * ZRA, ZRA), :]
        )

    return _agg_kernel


_agg128 = _make_agg_kernel(FIN)
_agg64 = _make_agg_kernel(CLS)


def _prep_body(f_ref, dp_ref, x1_ref, ns_ref, nd_ref):
    dout = dp_ref[0, 0] + dp_ref[1, 0]
    din = dp_ref[0, 1] + dp_ref[1, 1]
    ns = lax.rsqrt(jnp.maximum(dout, 1.0)).reshape(BR, 1)
    nd = lax.rsqrt(jnp.maximum(din, 1.0)).reshape(BR, 1)
    x1_ref[...] = f_ref[...] * ns
    ns_ref[...] = ns
    nd_ref[...] = nd


def _mid_body(p_ref, nd_ref, ns_ref, w1_ref, b1_ref, w2_ref, t2_ref):
    agg = (p_ref[0] + p_ref[1]) * nd_ref[...]
    h = jnp.dot(agg, w1_ref[...], preferred_element_type=jnp.float32)
    h = jnp.maximum(h + b1_ref[...], 0.0)
    t2_ref[...] = jnp.dot(
        h * ns_ref[...], w2_ref[...], preferred_element_type=jnp.float32
    )


def _fin_body(p_ref, nd_ref, b2_ref, out_ref):
    sc = (p_ref[0] + p_ref[1]) * nd_ref[...] + b2_ref[...]
    m = jnp.max(sc, axis=1, keepdims=True)
    lse = jnp.log(jnp.sum(jnp.exp(sc - m), axis=1, keepdims=True)) + m
    out_ref[...] = sc - lse


def _row_block(shape_minor):
    return pl.BlockSpec((BR,) + shape_minor, lambda i: (i,) + (0,) * len(shape_minor))


def kernel(features, edge_index, W1, b1, W2, b2):
    pad = (N + jnp.arange(EPAD, dtype=jnp.int32) % (NACC - N))
    src2 = jnp.concatenate([edge_index[0], pad]).reshape(ROWS, CH)
    dst2 = jnp.concatenate([edge_index[1], pad]).reshape(ROWS, CH)

    deg = _deg_kernel(src2, dst2)

    x1, ns, nd = pl.pallas_call(
        _prep_body,
        grid=(GRID,),
        in_specs=[
            _row_block((FIN,)),
            pl.BlockSpec((NC, 2, BR), lambda i: (0, 0, i)),
        ],
        out_specs=[_row_block((FIN,)), _row_block((1,)), _row_block((1,))],
        out_shape=[
            jax.ShapeDtypeStruct((NP, FIN), jnp.float32),
            jax.ShapeDtypeStruct((NP, 1), jnp.float32),
            jax.ShapeDtypeStruct((NP, 1), jnp.float32),
        ],
    )(features, deg)

    p1 = _agg128(x1, src2, dst2)

    t2 = pl.pallas_call(
        _mid_body,
        grid=(GRID,),
        in_specs=[
            pl.BlockSpec((NC, BR, FIN), lambda i: (0, i, 0)),
            _row_block((1,)),
            _row_block((1,)),
            pl.BlockSpec((FIN, HID), lambda i: (0, 0)),
            pl.BlockSpec((1, HID), lambda i: (0, 0)),
            pl.BlockSpec((HID, CLS), lambda i: (0, 0)),
        ],
        out_specs=_row_block((CLS,)),
        out_shape=jax.ShapeDtypeStruct((NP, CLS), jnp.float32),
    )(p1, nd, ns, W1, b1.reshape(1, HID), W2)

    p2 = _agg64(t2, src2, dst2)

    out = pl.pallas_call(
        _fin_body,
        grid=(GRID,),
        in_specs=[
            pl.BlockSpec((NC, BR, CLS), lambda i: (0, i, 0)),
            _row_block((1,)),
            pl.BlockSpec((1, CLS), lambda i: (0, 0)),
        ],
        out_specs=_row_block((CLS,)),
        out_shape=jax.ShapeDtypeStruct((NP, CLS), jnp.float32),
    )(p2, nd, b2.reshape(1, CLS))

    return out[:N]


# speedup vs baseline: 1.1125x; 1.0925x over previous
import functools

import jax
import jax.numpy as jnp
from jax import lax
from jax.experimental import pallas as pl
from jax.experimental.pallas import tpu as pltpu
from jax.experimental.pallas import tpu_sc as plsc

N = 10000
E = 320000
FIN = 128
HID = 128
CLS = 64

NC = 2
NS = 16
NW = NC * NS
CH = 128
ROWS = E // CH
NP = 10240
NACC = 10112
ZR = NP // NS
ZRA = NACC // NS

BR = 2048
GRID = NP // BR

_mesh = plsc.VectorSubcoreMesh(
    core_axis_name="c", subcore_axis_name="s", num_cores=NC, num_subcores=NS
)


NCH = 80
LASTW = NW - 1
NCHL = ROWS - NCH * LASTW
NBUF = 5


def _worker_ids():
    c = lax.axis_index("c")
    s = lax.axis_index("s")
    return c, s, c * NS + s


def _num_chunks(w):
    return jnp.where(w == LASTW, NCHL, NCH)


def _load_idx(src2, dst2, srcall, dstall, w):
    base = pl.multiple_of(w * NCH, 8)

    @pl.when(w < LASTW)
    def _():
        pltpu.sync_copy(src2.at[pl.ds(base, NCH)], srcall)
        pltpu.sync_copy(dst2.at[pl.ds(base, NCH)], dstall)

    @pl.when(w == LASTW)
    def _():
        pltpu.sync_copy(
            src2.at[pl.ds(NCH * LASTW, NCHL)], srcall.at[pl.ds(0, NCHL)]
        )
        pltpu.sync_copy(
            dst2.at[pl.ds(NCH * LASTW, NCHL)], dstall.at[pl.ds(0, NCHL)]
        )


def _fill_zeros_2d(ref, nrows, ncols):
    z = jnp.zeros((16,), jnp.float32)

    def body(i, carry):
        r = i // (ncols // 16)
        col = (i % (ncols // 16)) * 16
        ref[r, pl.ds(col, 16)] = z
        return carry

    lax.fori_loop(0, nrows * ncols // 16, body, 0)


def _fill_1d(ref, n, value):
    v = jnp.full((16,), value, jnp.float32)

    def body(i, carry):
        ref[pl.ds(i * 16, 16)] = v
        return carry

    lax.fori_loop(0, n // 16, body, 0)


@functools.partial(
    pl.kernel,
    out_type=jax.ShapeDtypeStruct((NC, 2, NP), jnp.float32),
    mesh=_mesh,
    compiler_params=pltpu.CompilerParams(skip_device_barrier=True),
    scratch_types=[
        pltpu.VMEM((NCH, CH), jnp.int32),
        pltpu.VMEM((NCH, CH), jnp.int32),
        pltpu.VMEM((CH,), jnp.float32),
        pltpu.VMEM((ZR,), jnp.float32),
        pltpu.VMEM_SHARED((NP,), jnp.float32),
        pltpu.VMEM_SHARED((NP,), jnp.float32),
    ]
    + [pltpu.SemaphoreType.DMA] * (2 * NBUF),
)
def _deg_kernel(src2, dst2, out, srcall, dstall, onesv, zbuf, acc_out,
                acc_in, *sems):
    sa, sb = sems[:NBUF], sems[NBUF:]
    c, s, w = _worker_ids()
    _load_idx(src2, dst2, srcall, dstall, w)
    _fill_1d(zbuf, ZR, 0.0)
    _fill_1d(onesv, CH, 1.0)
    zb = s * ZR
    pltpu.sync_copy(zbuf, acc_out.at[pl.ds(zb, ZR)])
    pltpu.sync_copy(zbuf, acc_in.at[pl.ds(zb, ZR)])
    plsc.subcore_barrier()

    def step(t, carry):
        for j in range(NBUF):
            k = t * NBUF + j
            pltpu.async_copy(onesv, acc_out.at[srcall.at[k]], sa[j], add=True)
            pltpu.async_copy(onesv, acc_in.at[dstall.at[k]], sb[j], add=True)

            @pl.when(t > 0)
            def _():
                pltpu.make_async_copy(onesv, acc_out.at[srcall.at[0]], sa[j]).wait()
                pltpu.make_async_copy(onesv, acc_in.at[dstall.at[0]], sb[j]).wait()

        return carry

    lax.fori_loop(0, _num_chunks(w) // NBUF, step, 0)
    for j in range(NBUF):
        pltpu.make_async_copy(onesv, acc_out.at[srcall.at[0]], sa[j]).wait()
        pltpu.make_async_copy(onesv, acc_in.at[dstall.at[0]], sb[j]).wait()

    plsc.subcore_barrier()
    pltpu.sync_copy(acc_out.at[pl.ds(zb, ZR)], out.at[c, 0, pl.ds(zb, ZR)])
    pltpu.sync_copy(acc_in.at[pl.ds(zb, ZR)], out.at[c, 1, pl.ds(zb, ZR)])


def _zero_acc(acc, zrows, s, nr):
    done = 0
    while done < nr:
        n = min(CH, nr - done)
        pltpu.sync_copy(
            zrows.at[pl.ds(0, n), :], acc.at[pl.ds(s * nr + done, n), :]
        )
        done += n


def _make_agg_kernel_slim(F):
    @functools.partial(
        pl.kernel,
        out_type=jax.ShapeDtypeStruct((NC, NACC, F), jnp.float32),
        mesh=_mesh,
        compiler_params=pltpu.CompilerParams(
            use_tc_tiling_on_sc=True, skip_device_barrier=True
        ),
        scratch_types=[pltpu.VMEM_SHARED((NACC, F), jnp.float32)]
        + [pltpu.VMEM((CH,), jnp.int32)] * 6
        + [pltpu.VMEM((CH, F), jnp.float32)] * 3
        + [pltpu.SemaphoreType.DMA] * 12,
    )
    def _agg_kernel(table, src2, dst2, out, acc, *ring):
        srcv, dstv = ring[0:3], ring[3:6]
        rows = ring[6:9]
        gs, ss = ring[9:12], ring[12:15]
        is_, id_ = ring[15:18], ring[18:21]
        c, s, w = _worker_ids()
        _fill_zeros_2d(rows[0], CH, F)
        _zero_acc(acc, rows[0], s, ZRA)
        plsc.subcore_barrier()

        def is_issue(k, b):
            pltpu.async_copy(src2.at[w * NCH + k], srcv[b], is_[b])

        def is_wait(b):
            pltpu.make_async_copy(src2.at[0], srcv[b], is_[b]).wait()

        def id_issue(k, b):
            pltpu.async_copy(dst2.at[w * NCH + k], dstv[b], id_[b])

        def id_wait(b):
            pltpu.make_async_copy(dst2.at[0], dstv[b], id_[b]).wait()

        def g_issue(b):
            pltpu.async_copy(table.at[srcv[b]], rows[b], gs[b])

        def g_wait(b):
            pltpu.make_async_copy(table.at[srcv[b]], rows[b], gs[b]).wait()

        def s_issue(b):
            pltpu.async_copy(rows[b], acc.at[dstv[b]], ss[b], add=True)

        def s_wait(b):
            pltpu.make_async_copy(rows[b], acc.at[dstv[b]], ss[b]).wait()

        def it(k, b, first=False, n2=True, n3=True):
            b2 = (b + 2) % 3
            id_wait(b)
            g_wait(b)
            s_issue(b)
            if not first:
                s_wait(b2)
            if n2:
                id_issue(k + 2, b2)
                is_wait(b2)
                g_issue(b2)
            if n3:
                is_issue(k + 3, b)

        is_issue(0, 0)
        id_issue(0, 0)
        is_issue(1, 1)
        id_issue(1, 1)
        is_issue(2, 2)
        is_wait(0)
        g_issue(0)
        is_wait(1)
        g_issue(1)
        it(0, 0, first=True)
        it(1, 1)

        def step(t, carry):
            for j in range(3):
                it(2 + 3 * t + j, (2 + j) % 3)
            return carry

        nchw = _num_chunks(w)
        lax.fori_loop(0, (nchw - 5) // 3, step, 0)
        it(nchw - 3, (NCH - 3) % 3, n3=False)
        it(nchw - 2, (NCH - 2) % 3, n2=False, n3=False)
        it(nchw - 1, (NCH - 1) % 3, n2=False, n3=False)
        s_wait((NCH - 1) % 3)
        plsc.subcore_barrier()
        pltpu.sync_copy(
            acc.at[pl.ds(s * ZRA, ZRA), :], out.at[c, pl.ds(s * ZRA, ZRA), :]
        )

    return _agg_kernel


def _make_agg_kernel(F):
    if F == FIN:
        return _make_agg_kernel_slim(F)

    @functools.partial(
        pl.kernel,
        out_type=jax.ShapeDtypeStruct((NC, NACC, F), jnp.float32),
        mesh=_mesh,
        compiler_params=pltpu.CompilerParams(
            use_tc_tiling_on_sc=(F % 128 == 0), skip_device_barrier=True
        ),
        scratch_types=[
            pltpu.VMEM((NCH, CH), jnp.int32),
            pltpu.VMEM((NCH, CH), jnp.int32),
            pltpu.VMEM_SHARED((NACC, F), jnp.float32),
        ]
        + [pltpu.VMEM((CH, F), jnp.float32)] * NBUF
        + [pltpu.SemaphoreType.DMA] * (2 * NBUF),
    )
    def _agg_kernel(table, src2, dst2, out, srcall, dstall, acc, *ring):
        rows = ring[:NBUF]
        gs = ring[NBUF:2 * NBUF]
        ss = ring[2 * NBUF:]
        c, s, w = _worker_ids()
        _load_idx(src2, dst2, srcall, dstall, w)
        _fill_zeros_2d(rows[0], CH, F)
        _zero_acc(acc, rows[0], s, ZRA)
        plsc.subcore_barrier()

        def g_issue(k, b):
            pltpu.async_copy(table.at[srcall.at[k]], rows[b], gs[b])

        def g_wait(b):
            pltpu.make_async_copy(table.at[srcall.at[0]], rows[b], gs[b]).wait()

        def s_issue(k, b):
            pltpu.async_copy(rows[b], acc.at[dstall.at[k]], ss[b], add=True)

        def s_wait(b):
            pltpu.make_async_copy(rows[b], acc.at[dstall.at[0]], ss[b]).wait()

        def iter_chunk(k, j, ss_wait, g_next):
            g_wait(j)
            s_issue(k, j)
            bn = (j + 3) % NBUF
            if ss_wait:
                s_wait(bn)
            if g_next:
                g_issue(k + 3, bn)

        for b in range(3):
            g_issue(b, b)
        for j in range(NBUF):
            iter_chunk(j, j, ss_wait=(j >= NBUF - 3), g_next=True)

        def step(t, carry):
            for j in range(NBUF):
                iter_chunk(t * NBUF + j, j, ss_wait=True, g_next=True)
            return carry

        nstepw = _num_chunks(w) // NBUF
        lax.fori_loop(1, nstepw - 1, step, 0)
        for j in range(NBUF):
            iter_chunk((nstepw - 1) * NBUF + j, j, ss_wait=(j < NBUF - 3),
                       g_next=(j < NBUF - 3))
        for b in range(NBUF):
            s_wait(b)

        plsc.subcore_barrier()
        pltpu.sync_copy(
            acc.at[pl.ds(s * ZRA, ZRA), :], out.at[c, pl.ds(s * ZRA, ZRA), :]
        )

    return _agg_kernel


_agg128 = _make_agg_kernel(FIN)
_agg64 = _make_agg_kernel(CLS)


def _prep_body(f_ref, dp_ref, x1_ref, ns_ref, nd_ref):
    dout = dp_ref[0, 0] + dp_ref[1, 0]
    din = dp_ref[0, 1] + dp_ref[1, 1]
    ns = lax.rsqrt(jnp.maximum(dout, 1.0)).reshape(BR, 1)
    nd = lax.rsqrt(jnp.maximum(din, 1.0)).reshape(BR, 1)
    x1_ref[...] = f_ref[...] * ns
    ns_ref[...] = ns
    nd_ref[...] = nd


def _mid_body(p_ref, nd_ref, ns_ref, w1_ref, b1_ref, w2_ref, t2_ref):
    agg = (p_ref[0] + p_ref[1]) * nd_ref[...]
    h = jnp.dot(agg, w1_ref[...], preferred_element_type=jnp.float32)
    h = jnp.maximum(h + b1_ref[...], 0.0)
    t2_ref[...] = jnp.dot(
        h * ns_ref[...], w2_ref[...], preferred_element_type=jnp.float32
    )


def _fin_body(p_ref, nd_ref, b2_ref, out_ref):
    sc = (p_ref[0] + p_ref[1]) * nd_ref[...] + b2_ref[...]
    m = jnp.max(sc, axis=1, keepdims=True)
    lse = jnp.log(jnp.sum(jnp.exp(sc - m), axis=1, keepdims=True)) + m
    out_ref[...] = sc - lse


def _row_block(shape_minor):
    return pl.BlockSpec((BR,) + shape_minor, lambda i: (i,) + (0,) * len(shape_minor))


def kernel(features, edge_index, W1, b1, W2, b2):
    src2 = edge_index[0].reshape(ROWS, CH)
    dst2 = edge_index[1].reshape(ROWS, CH)

    deg = _deg_kernel(src2, dst2)

    x1, ns, nd = pl.pallas_call(
        _prep_body,
        grid=(GRID,),
        in_specs=[
            _row_block((FIN,)),
            pl.BlockSpec((NC, 2, BR), lambda i: (0, 0, i)),
        ],
        out_specs=[_row_block((FIN,)), _row_block((1,)), _row_block((1,))],
        out_shape=[
            jax.ShapeDtypeStruct((NP, FIN), jnp.float32),
            jax.ShapeDtypeStruct((NP, 1), jnp.float32),
            jax.ShapeDtypeStruct((NP, 1), jnp.float32),
        ],
    )(features, deg)

    p1 = _agg128(x1, src2, dst2)

    t2 = pl.pallas_call(
        _mid_body,
        grid=(GRID,),
        in_specs=[
            pl.BlockSpec((NC, BR, FIN), lambda i: (0, i, 0)),
            _row_block((1,)),
            _row_block((1,)),
            pl.BlockSpec((FIN, HID), lambda i: (0, 0)),
            pl.BlockSpec((1, HID), lambda i: (0, 0)),
            pl.BlockSpec((HID, CLS), lambda i: (0, 0)),
        ],
        out_specs=_row_block((CLS,)),
        out_shape=jax.ShapeDtypeStruct((NP, CLS), jnp.float32),
    )(p1, nd, ns, W1, b1.reshape(1, HID), W2)

    p2 = _agg64(t2, src2, dst2)

    out = pl.pallas_call(
        _fin_body,
        grid=(GRID,),
        in_specs=[
            pl.BlockSpec((NC, BR, CLS), lambda i: (0, i, 0)),
            _row_block((1,)),
            pl.BlockSpec((1, CLS), lambda i: (0, 0)),
        ],
        out_specs=_row_block((CLS,)),
        out_shape=jax.ShapeDtypeStruct((N, CLS), jnp.float32),
    )(p2, nd, b2.reshape(1, CLS))

    return out
